# baseline (speedup 1.0000x reference)
import jax
import jax.numpy as jnp
from jax.experimental import pallas as pl
from jax.experimental.pallas import tpu as pltpu

H = 768
HD = H // 2
B, T = 8, 512
G4 = 4 * HD
BT = 32
NBLK = T // BT
UNROLL = 32


def _dot(a, b):
    return jnp.dot(a, b, preferred_element_type=jnp.float32)


def _lstm_cell(gates, h, c, whh_ref):
    g = gates + _dot(h.astype(jnp.bfloat16), whh_ref[...])
    ig = jax.nn.sigmoid(g[:, 0:HD])
    fg = jax.nn.sigmoid(g[:, HD:2 * HD])
    gg = jnp.tanh(g[:, 2 * HD:3 * HD])
    og = jax.nn.sigmoid(g[:, 3 * HD:])
    c = fg * c + ig * gg
    h = og * jnp.tanh(c)
    return h, c


def _layer0_kernel(xf_ref, xr_ref, wihf_ref, whhf_ref, bf_ref,
                   wihr_ref, whhr_ref, br_ref,
                   outf_ref, outr_ref,
                   hf_s, cf_s, hr_s, cr_s):
    i = pl.program_id(0)

    @pl.when(i == 0)
    def _init():
        hf_s[...] = jnp.zeros_like(hf_s)
        cf_s[...] = jnp.zeros_like(cf_s)
        hr_s[...] = jnp.zeros_like(hr_s)
        cr_s[...] = jnp.zeros_like(cr_s)

    xf = xf_ref[...].reshape(BT * B, H).astype(jnp.bfloat16)
    xr = xr_ref[...].reshape(BT * B, H).astype(jnp.bfloat16)
    gf = _dot(xf, wihf_ref[...]) + bf_ref[...]
    gr = _dot(xr, wihr_ref[...]) + br_ref[...]

    hf, cf, hr, cr = hf_s[...], cf_s[...], hr_s[...], cr_s[...]
    for s in range(BT):
        sr = BT - 1 - s
        hf, cf = _lstm_cell(gf[s * B:(s + 1) * B], hf, cf, whhf_ref)
        outf_ref[s:s + 1] = hf[None]
        hr, cr = _lstm_cell(gr[sr * B:(sr + 1) * B], hr, cr, whhr_ref)
        outr_ref[sr:sr + 1] = hr[None]
    hf_s[...], cf_s[...], hr_s[...], cr_s[...] = hf, cf, hr, cr


def _layer1_kernel(af_ref, bf_ref, ar_ref, br_ref,
                   wihf_a_ref, wihf_b_ref, whhf_ref, biasf_ref,
                   wihr_a_ref, wihr_b_ref, whhr_ref, biasr_ref,
                   node_ref,
                   hf_s, cf_s, hr_s, cr_s, accf_s, accr_s):
    i = pl.program_id(0)

    @pl.when(i == 0)
    def _init():
        hf_s[...] = jnp.zeros_like(hf_s)
        cf_s[...] = jnp.zeros_like(cf_s)
        hr_s[...] = jnp.zeros_like(hr_s)
        cr_s[...] = jnp.zeros_like(cr_s)
        accf_s[...] = jnp.zeros_like(accf_s)
        accr_s[...] = jnp.zeros_like(accr_s)

    af = af_ref[...].reshape(BT * B, HD).astype(jnp.bfloat16)
    bf = bf_ref[...].reshape(BT * B, HD).astype(jnp.bfloat16)
    ar = ar_ref[...].reshape(BT * B, HD).astype(jnp.bfloat16)
    br = br_ref[...].reshape(BT * B, HD).astype(jnp.bfloat16)
    gf = _dot(af, wihf_a_ref[...]) + _dot(bf, wihf_b_ref[...]) + biasf_ref[...]
    gr = _dot(ar, wihr_a_ref[...]) + _dot(br, wihr_b_ref[...]) + biasr_ref[...]

    hf, cf, hr, cr = hf_s[...], cf_s[...], hr_s[...], cr_s[...]
    accf, accr = accf_s[...], accr_s[...]
    for s in range(BT):
        sr = BT - 1 - s
        hf, cf = _lstm_cell(gf[s * B:(s + 1) * B], hf, cf, whhf_ref)
        hr, cr = _lstm_cell(gr[sr * B:(sr + 1) * B], hr, cr, whhr_ref)
        accf = accf + hf
        accr = accr + hr
    hf_s[...], cf_s[...], hr_s[...], cr_s[...] = hf, cf, hr, cr
    accf_s[...], accr_s[...] = accf, accr

    @pl.when(i == NBLK - 1)
    def _emit():
        inv_t = jnp.float32(1.0 / T)
        node_ref[:, 0:HD] = accf_s[...] * inv_t
        node_ref[:, HD:H] = accr_s[...] * inv_t


def _fwd_map(i):
    return (i, 0, 0)


def _rev_map(i):
    return (NBLK - 1 - i, 0, 0)


def _full_map2(i):
    return (0, 0)


def kernel(context_hidden,
           W_ih_l0, W_hh_l0, b_ih_l0, b_hh_l0,
           W_ih_l0_r, W_hh_l0_r, b_ih_l0_r, b_hh_l0_r,
           W_ih_l1, W_hh_l1, b_ih_l1, b_hh_l1,
           W_ih_l1_r, W_hh_l1_r, b_ih_l1_r, b_hh_l1_r):
    f32 = jnp.float32
    bf16 = jnp.bfloat16
    x = jnp.swapaxes(context_hidden, 0, 1)

    seq_spec_f = pl.BlockSpec((BT, B, H), _fwd_map)
    seq_spec_r = pl.BlockSpec((BT, B, H), _rev_map)
    hd_spec_f = pl.BlockSpec((BT, B, HD), _fwd_map)
    hd_spec_r = pl.BlockSpec((BT, B, HD), _rev_map)

    def wspec(shape):
        return pl.BlockSpec(shape, _full_map2)

    cparams = pltpu.CompilerParams(dimension_semantics=("arbitrary",))

    wihf0 = W_ih_l0.T.astype(bf16)
    wihr0 = W_ih_l0_r.T.astype(bf16)
    whhf0 = W_hh_l0.T.astype(bf16)
    whhr0 = W_hh_l0_r.T.astype(bf16)
    bf0 = (b_ih_l0 + b_hh_l0).reshape(1, G4)
    br0 = (b_ih_l0_r + b_hh_l0_r).reshape(1, G4)

    hs_f, hs_r = pl.pallas_call(
        _layer0_kernel,
        grid=(NBLK,),
        in_specs=[seq_spec_f, seq_spec_r,
                  wspec((H, G4)), wspec((HD, G4)), wspec((1, G4)),
                  wspec((H, G4)), wspec((HD, G4)), wspec((1, G4))],
        out_specs=[hd_spec_f, hd_spec_r],
        out_shape=[jax.ShapeDtypeStruct((T, B, HD), f32),
                   jax.ShapeDtypeStruct((T, B, HD), f32)],
        scratch_shapes=[pltpu.VMEM((B, HD), f32)] * 4,
        compiler_params=cparams,
    )(x, x, wihf0, whhf0, bf0, wihr0, whhr0, br0)

    wihf1 = W_ih_l1.T.astype(bf16)
    wihr1 = W_ih_l1_r.T.astype(bf16)
    whhf1 = W_hh_l1.T.astype(bf16)
    whhr1 = W_hh_l1_r.T.astype(bf16)
    bf1 = (b_ih_l1 + b_hh_l1).reshape(1, G4)
    br1 = (b_ih_l1_r + b_hh_l1_r).reshape(1, G4)

    node = pl.pallas_call(
        _layer1_kernel,
        grid=(NBLK,),
        in_specs=[pl.BlockSpec((BT, B, HD), _fwd_map),
                  pl.BlockSpec((BT, B, HD), _fwd_map),
                  pl.BlockSpec((BT, B, HD), _rev_map),
                  pl.BlockSpec((BT, B, HD), _rev_map),
                  wspec((HD, G4)), wspec((HD, G4)), wspec((HD, G4)),
                  wspec((1, G4)),
                  wspec((HD, G4)), wspec((HD, G4)), wspec((HD, G4)),
                  wspec((1, G4))],
        out_specs=pl.BlockSpec((B, H), _full_map2),
        out_shape=jax.ShapeDtypeStruct((B, H), f32),
        scratch_shapes=[pltpu.VMEM((B, HD), f32)] * 6,
        compiler_params=cparams,
    )(hs_f, hs_r, hs_f, hs_r,
      wihf1[:HD], wihf1[HD:], whhf1, bf1,
      wihr1[:HD], wihr1[HD:], whhr1, br1)

    edge_index = jnp.array([[0, 1], [1, 0]], dtype=jnp.int32)
    edge_types = jnp.array([0, 0], dtype=jnp.int32)
    return node, edge_index, edge_types


# speedup vs baseline: 1.0022x; 1.0022x over previous
import jax
import jax.numpy as jnp
from jax.experimental import pallas as pl
from jax.experimental.pallas import tpu as pltpu

H = 768
HD = H // 2
B, T = 8, 512
G4 = 4 * HD
BT = 32
NBLK = T // BT


def _dot(a, b):
    return jnp.dot(a, b, preferred_element_type=jnp.float32)


def _lstm_cell(gates, h, c, whh_ref):
    g = gates + _dot(h.astype(jnp.bfloat16), whh_ref[...])
    ig = jax.nn.sigmoid(g[:, 0:HD])
    fg = jax.nn.sigmoid(g[:, HD:2 * HD])
    gg = jnp.tanh(g[:, 2 * HD:3 * HD])
    og = jax.nn.sigmoid(g[:, 3 * HD:])
    c = fg * c + ig * gg
    h = og * jnp.tanh(c)
    return h, c


def _fused_kernel(xf_ref, xr_ref,
                  wihf0_ref, whhf0_ref, bf0_ref,
                  wihr0_ref, whhr0_ref, br0_ref,
                  w1fa_ref, w1fb_ref, whhf1_ref, bf1_ref,
                  w1ra_ref, w1rb_ref, whhr1_ref, br1_ref,
                  node_ref,
                  hsf_s, hsr_s,
                  hf_s, cf_s, hr_s, cr_s, accf_s, accr_s):
    i = pl.program_id(0)

    @pl.when((i == 0) | (i == NBLK))
    def _init():
        hf_s[...] = jnp.zeros_like(hf_s)
        cf_s[...] = jnp.zeros_like(cf_s)
        hr_s[...] = jnp.zeros_like(hr_s)
        cr_s[...] = jnp.zeros_like(cr_s)
        accf_s[...] = jnp.zeros_like(accf_s)
        accr_s[...] = jnp.zeros_like(accr_s)

    @pl.when(i < NBLK)
    def _layer0():
        xf = xf_ref[...].reshape(BT * B, H).astype(jnp.bfloat16)
        xr = xr_ref[...].reshape(BT * B, H).astype(jnp.bfloat16)
        gf = _dot(xf, wihf0_ref[...]) + bf0_ref[...]
        gr = _dot(xr, wihr0_ref[...]) + br0_ref[...]

        base_f = i * BT
        base_r = (NBLK - 1 - i) * BT
        hf, cf, hr, cr = hf_s[...], cf_s[...], hr_s[...], cr_s[...]
        for s in range(BT):
            sr = BT - 1 - s
            hf, cf = _lstm_cell(gf[s * B:(s + 1) * B], hf, cf, whhf0_ref)
            hsf_s[pl.ds(base_f + s, 1)] = hf[None]
            hr, cr = _lstm_cell(gr[sr * B:(sr + 1) * B], hr, cr, whhr0_ref)
            hsr_s[pl.ds(base_r + sr, 1)] = hr[None]
        hf_s[...], cf_s[...], hr_s[...], cr_s[...] = hf, cf, hr, cr

    @pl.when(i >= NBLK)
    def _layer1():
        j = i - NBLK
        base_f = j * BT
        base_r = (NBLK - 1 - j) * BT
        af = hsf_s[pl.ds(base_f, BT)].reshape(BT * B, HD).astype(jnp.bfloat16)
        bf = hsr_s[pl.ds(base_f, BT)].reshape(BT * B, HD).astype(jnp.bfloat16)
        ar = hsf_s[pl.ds(base_r, BT)].reshape(BT * B, HD).astype(jnp.bfloat16)
        br = hsr_s[pl.ds(base_r, BT)].reshape(BT * B, HD).astype(jnp.bfloat16)
        gf = _dot(af, w1fa_ref[...]) + _dot(bf, w1fb_ref[...]) + bf1_ref[...]
        gr = _dot(ar, w1ra_ref[...]) + _dot(br, w1rb_ref[...]) + br1_ref[...]

        hf, cf, hr, cr = hf_s[...], cf_s[...], hr_s[...], cr_s[...]
        accf, accr = accf_s[...], accr_s[...]
        for s in range(BT):
            sr = BT - 1 - s
            hf, cf = _lstm_cell(gf[s * B:(s + 1) * B], hf, cf, whhf1_ref)
            hr, cr = _lstm_cell(gr[sr * B:(sr + 1) * B], hr, cr, whhr1_ref)
            accf = accf + hf
            accr = accr + hr
        hf_s[...], cf_s[...], hr_s[...], cr_s[...] = hf, cf, hr, cr
        accf_s[...], accr_s[...] = accf, accr

    @pl.when(i == 2 * NBLK - 1)
    def _emit():
        inv_t = jnp.float32(1.0 / T)
        node_ref[:, 0:HD] = accf_s[...] * inv_t
        node_ref[:, HD:H] = accr_s[...] * inv_t


def _fwd_map(i):
    return (jnp.minimum(i, NBLK - 1), 0, 0)


def _rev_map(i):
    return (jnp.maximum(NBLK - 1 - i, 0), 0, 0)


def _full_map2(i):
    return (0, 0)


def kernel(context_hidden,
           W_ih_l0, W_hh_l0, b_ih_l0, b_hh_l0,
           W_ih_l0_r, W_hh_l0_r, b_ih_l0_r, b_hh_l0_r,
           W_ih_l1, W_hh_l1, b_ih_l1, b_hh_l1,
           W_ih_l1_r, W_hh_l1_r, b_ih_l1_r, b_hh_l1_r):
    f32 = jnp.float32
    bf16 = jnp.bfloat16
    x = jnp.swapaxes(context_hidden, 0, 1)

    def wspec(shape):
        return pl.BlockSpec(shape, _full_map2)

    wihf0 = W_ih_l0.T.astype(bf16)
    wihr0 = W_ih_l0_r.T.astype(bf16)
    whhf0 = W_hh_l0.T.astype(bf16)
    whhr0 = W_hh_l0_r.T.astype(bf16)
    bf0 = (b_ih_l0 + b_hh_l0).reshape(1, G4)
    br0 = (b_ih_l0_r + b_hh_l0_r).reshape(1, G4)
    wihf1 = W_ih_l1.T.astype(bf16)
    wihr1 = W_ih_l1_r.T.astype(bf16)
    whhf1 = W_hh_l1.T.astype(bf16)
    whhr1 = W_hh_l1_r.T.astype(bf16)
    bf1 = (b_ih_l1 + b_hh_l1).reshape(1, G4)
    br1 = (b_ih_l1_r + b_hh_l1_r).reshape(1, G4)

    node = pl.pallas_call(
        _fused_kernel,
        grid=(2 * NBLK,),
        in_specs=[pl.BlockSpec((BT, B, H), _fwd_map),
                  pl.BlockSpec((BT, B, H), _rev_map),
                  wspec((H, G4)), wspec((HD, G4)), wspec((1, G4)),
                  wspec((H, G4)), wspec((HD, G4)), wspec((1, G4)),
                  wspec((HD, G4)), wspec((HD, G4)), wspec((HD, G4)),
                  wspec((1, G4)),
                  wspec((HD, G4)), wspec((HD, G4)), wspec((HD, G4)),
                  wspec((1, G4))],
        out_specs=pl.BlockSpec((B, H), _full_map2),
        out_shape=jax.ShapeDtypeStruct((B, H), f32),
        scratch_shapes=[pltpu.VMEM((T, B, HD), f32)] * 2
                       + [pltpu.VMEM((B, HD), f32)] * 6,
        compiler_params=pltpu.CompilerParams(
            dimension_semantics=("arbitrary",)),
    )(x, x, wihf0, whhf0, bf0, wihr0, whhr0, br0,
      wihf1[:HD], wihf1[HD:], whhf1, bf1,
      wihr1[:HD], wihr1[HD:], whhr1, br1)

    edge_index = jnp.array([[0, 1], [1, 0]], dtype=jnp.int32)
    edge_types = jnp.array([0, 0], dtype=jnp.int32)
    return node, edge_index, edge_types
